# baseline (speedup 1.0000x reference)
import jax
import jax.numpy as jnp
from jax import lax
from jax.experimental import pallas as pl
from jax.experimental.pallas import tpu as pltpu
from jax.experimental.pallas import tpu_sc as plsc

B = 64
Q = 900
C = 91
N = Q * C
NPAD = 81920
NVEC = NPAD // 16
NSEL = 300
NSEL_PAD = 304
CAP = 512
CBUF = CAP + 16
HBINS = 16384
BIG = 1 << 20


def _vec16(i):
    return pl.ds(pl.multiple_of(i * 16, 16), 16)


def _scan_top(hist, start_chunk, need):
    def cond(c):
        cum, _ = c
        return cum < need

    def body(c):
        cum, j = c
        h = hist[_vec16(j)]
        return cum + jnp.sum(h), j - 1

    cum, j = lax.while_loop(cond, body, (jnp.int32(0), start_chunk))
    jc = j + 1
    h = hist[_vec16(jc)]
    base = cum - jnp.sum(h)
    rev = lax.rev(h, (0,))
    cs = plsc.cumsum(rev)
    lanes = lax.iota(jnp.int32, 16)
    f = jnp.max(plsc.all_reduce_ffs((base + cs) >= need) * jnp.ones((16,), jnp.int32))
    hb = jnp.sum(jnp.where(lanes == f, rev, 0))
    csf = jnp.sum(jnp.where(lanes == f, cs, 0))
    return jc * 16 + 15 - f, base + csf - hb, hb


def _body(prob_hbm, boxes_hbm, ts_hbm, out_boxes, out_scores, out_labels,
          prob_buf, hist, ckey, cidx, boxf, ts_buf,
          boxes_st, scores_st, labels_st, qidx_st):
    wid = lax.axis_index("s") * 2 + lax.axis_index("c")
    lanes = lax.iota(jnp.int32, 16)
    ones = jnp.ones((16,), jnp.int32)
    zeros = jnp.zeros((16,), jnp.int32)

    pltpu.sync_copy(ts_hbm, ts_buf)

    for sub in range(2):
        b = wid * 2 + sub
        prob_buf[_vec16(NVEC - 2)] = jnp.zeros((16,), jnp.float32)
        prob_buf[_vec16(NVEC - 1)] = jnp.zeros((16,), jnp.float32)
        pltpu.sync_copy(prob_hbm.at[b], prob_buf.at[pl.ds(0, N)])
        pltpu.sync_copy(boxes_hbm.at[b], boxf)

        def zh(i, _):
            hist[_vec16(i)] = zeros
            return 0

        lax.fori_loop(0, HBINS // 16, zh, 0, unroll=8)

        def hb1(i, m):
            v = prob_buf[_vec16(i)]
            bits = lax.bitcast_convert_type(v, jnp.int32)
            plsc.addupdate_scatter(hist, [lax.shift_right_logical(bits, 16)], ones)
            return jnp.maximum(m, v)

        maxv = lax.fori_loop(0, NVEC, hb1, jnp.zeros((16,), jnp.float32),
                             unroll=8)
        maxbin = jnp.max(lax.shift_right_logical(
            lax.bitcast_convert_type(maxv, jnp.int32), 16))

        bin1, above1, h1 = _scan_top(hist, lax.shift_right_logical(maxbin, 4),
                                     jnp.int32(NSEL))

        def sub_hist(pshift, pval, oshift):
            lax.fori_loop(0, 16, zh, 0)

            def hb2(i, _):
                v = prob_buf[_vec16(i)]
                bits = lax.bitcast_convert_type(v, jnp.int32)
                sel = lax.shift_right_logical(bits, pshift) == pval
                sbin = lax.shift_right_logical(bits, oshift) & 0xFF
                plsc.addupdate_scatter(hist, [sbin], ones, mask=sel)
                return 0

            lax.fori_loop(0, NVEC, hb2, 0, unroll=4)

        def case_a():
            return bin1 << 16, jnp.int32(BIG)

        def case_bc():
            sub_hist(16, bin1, 8)
            bin2, above2, h2 = _scan_top(hist, jnp.int32(15), NSEL - above1)

            def case_b():
                return (bin1 << 16) | (bin2 << 8), jnp.int32(BIG)

            def case_c():
                sub_hist(8, (bin1 << 8) | bin2, 0)
                bin3, above3, _ = _scan_top(hist, jnp.int32(15),
                                            NSEL - above1 - above2)
                cnt_gt = above1 + above2 + above3
                return (bin1 << 16) | (bin2 << 8) | bin3, NSEL - cnt_gt

            return lax.cond(above1 + above2 + h2 <= CAP, case_b, case_c)

        tlow_bits, need_eq = lax.cond(above1 + h1 <= CAP, case_a, case_bc)
        tlow_f = lax.bitcast_convert_type(jnp.broadcast_to(tlow_bits, (16,)),
                                          jnp.float32)

        for i in range(CBUF // 16):
            cidx[_vec16(i)] = jnp.full((16,), NPAD - 1, jnp.int32)

        def cb_fast(i, g):
            v = prob_buf[_vec16(i)]
            keep = v >= tlow_f
            plsc.store_compressed(cidx.at[pl.ds(g, 16)], i * 16 + lanes,
                                  mask=keep)
            return g + jnp.sum(keep.astype(jnp.int32))

        def cb_slow(i, carry):
            g, e = carry
            v = prob_buf[_vec16(i)]
            m_gt = v > tlow_f
            m_eq = v == tlow_f
            pfx = plsc.cumsum(m_eq.astype(jnp.int32))
            keep = jnp.logical_or(
                m_gt, jnp.logical_and(m_eq, (e + pfx) <= need_eq))
            plsc.store_compressed(cidx.at[pl.ds(g, 16)], i * 16 + lanes,
                                  mask=keep)
            return (g + jnp.sum(keep.astype(jnp.int32)),
                    e + jnp.sum(m_eq.astype(jnp.int32)))

        def compact_fast():
            return lax.fori_loop(0, NVEC, cb_fast, jnp.int32(0), unroll=4)

        def compact_slow():
            g, _ = lax.fori_loop(0, NVEC, cb_slow,
                                 (jnp.int32(0), jnp.int32(0)))
            return g

        g = lax.cond(need_eq >= BIG, compact_fast, compact_slow)
        gv = (g + 15) // 16

        def fill_keys(i, _):
            ckey[_vec16(i)] = plsc.load_gather(prob_buf, [cidx[_vec16(i)]])
            return 0

        lax.fori_loop(0, CBUF // 16, fill_keys, 0, unroll=4)

        def zq(r, _):
            qidx_st[_vec16(r)] = zeros
            return 0

        lax.fori_loop(0, NSEL_PAD // 16, zq, 0)

        def rank_outer(ev, _):
            key_e = ckey[_vec16(ev)]
            idx_e = cidx[_vec16(ev)]
            pos_e = ev * 16 + lanes

            def rchunk(jc, r):
                kj = ckey[_vec16(jc)]
                jbase = jc * 16
                for l in range(16):
                    kb = kj[jnp.full((16,), l, jnp.int32)]
                    gt = kb > key_e
                    eqb = jnp.logical_and(kb == key_e, jbase + l < pos_e)
                    r = r + jnp.logical_or(gt, eqb).astype(jnp.int32)
                return r

            rank = lax.fori_loop(0, gv, rchunk, zeros)
            msk = rank < NSEL
            plsc.store_scatter(scores_st, [rank], key_e, mask=msk)
            plsc.store_scatter(labels_st, [rank], idx_e % C, mask=msk)
            plsc.store_scatter(qidx_st, [rank], idx_e // C, mask=msk)
            return 0

        lax.fori_loop(0, gv, rank_outer, 0)

        img_h = plsc.load_gather(ts_buf, [jnp.broadcast_to(b, (16,)), zeros])
        img_w = plsc.load_gather(ts_buf, [jnp.broadcast_to(b, (16,)), ones])

        def bx(r, _):
            q = qidx_st[_vec16(r)]
            cx = plsc.load_gather(boxf, [q, zeros])
            cy = plsc.load_gather(boxf, [q, ones])
            w = plsc.load_gather(boxf, [q, ones + 1])
            h = plsc.load_gather(boxf, [q, ones + 2])
            rows = r * 16 + lanes
            plsc.store_scatter(boxes_st, [rows, zeros], (cx - 0.5 * w) * img_w)
            plsc.store_scatter(boxes_st, [rows, ones], (cy - 0.5 * h) * img_h)
            plsc.store_scatter(boxes_st, [rows, ones + 1], (cx + 0.5 * w) * img_w)
            plsc.store_scatter(boxes_st, [rows, ones + 2], (cy + 0.5 * h) * img_h)
            return 0

        lax.fori_loop(0, NSEL_PAD // 16, bx, 0)

        pltpu.sync_copy(boxes_st.at[pl.ds(0, NSEL)], out_boxes.at[b])
        pltpu.sync_copy(scores_st.at[pl.ds(0, NSEL)], out_scores.at[b])
        pltpu.sync_copy(labels_st.at[pl.ds(0, NSEL)], out_labels.at[b])


@jax.jit
def kernel(pred_logits, pred_boxes, target_sizes):
    prob = jax.nn.sigmoid(pred_logits).reshape(B, N)

    mesh = plsc.VectorSubcoreMesh(core_axis_name="c", subcore_axis_name="s",
                                  num_cores=2, num_subcores=16)
    run = pl.kernel(
        _body,
        out_type=[
            jax.ShapeDtypeStruct((B, NSEL, 4), jnp.float32),
            jax.ShapeDtypeStruct((B, NSEL), jnp.float32),
            jax.ShapeDtypeStruct((B, NSEL), jnp.int32),
        ],
        mesh=mesh,
        compiler_params=pltpu.CompilerParams(needs_layout_passes=False,
                                             use_tc_tiling_on_sc=False),
        scratch_types=[
            pltpu.VMEM((NPAD,), jnp.float32),
            pltpu.VMEM((HBINS,), jnp.int32),
            pltpu.VMEM((CBUF,), jnp.float32),
            pltpu.VMEM((CBUF,), jnp.int32),
            pltpu.VMEM((Q, 4), jnp.float32),
            pltpu.VMEM((B, 2), jnp.float32),
            pltpu.VMEM((NSEL_PAD, 4), jnp.float32),
            pltpu.VMEM((NSEL_PAD,), jnp.float32),
            pltpu.VMEM((NSEL_PAD,), jnp.int32),
            pltpu.VMEM((NSEL_PAD,), jnp.int32),
        ],
    )
    boxes, scores, labels = run(prob, pred_boxes, target_sizes)
    return boxes, scores, labels


# speedup vs baseline: 1.0003x; 1.0003x over previous
import jax
import jax.numpy as jnp
from jax import lax
from jax.experimental import pallas as pl
from jax.experimental.pallas import tpu as pltpu
from jax.experimental.pallas import tpu_sc as plsc

B = 64
Q = 900
C = 91
N = Q * C
NPAD = 81920
NVEC = NPAD // 16
NSEL = 300
NSEL_PAD = 304
CAP = 512
CBUF = CAP + 16
HBINS = 16384
BIG = 1 << 20


def _vec16(i):
    return pl.ds(pl.multiple_of(i * 16, 16), 16)


def _scan_top(hist, start_chunk, need):
    def cond(c):
        cum, _ = c
        return cum < need

    def body(c):
        cum, j = c
        h = hist[_vec16(j)]
        return cum + jnp.sum(h), j - 1

    cum, j = lax.while_loop(cond, body, (jnp.int32(0), start_chunk))
    jc = j + 1
    h = hist[_vec16(jc)]
    base = cum - jnp.sum(h)
    rev = lax.rev(h, (0,))
    cs = plsc.cumsum(rev)
    lanes = lax.iota(jnp.int32, 16)
    f = jnp.max(plsc.all_reduce_ffs((base + cs) >= need) * jnp.ones((16,), jnp.int32))
    hb = jnp.sum(jnp.where(lanes == f, rev, 0))
    csf = jnp.sum(jnp.where(lanes == f, cs, 0))
    return jc * 16 + 15 - f, base + csf - hb, hb


def _body(prob_hbm, boxes_hbm, ts_hbm, out_boxes, out_scores, out_labels,
          prob_buf, hist, ckey, cidx, boxf, ts_buf,
          boxes_st, scores_st, labels_st, qidx_st):
    wid = lax.axis_index("s") * 2 + lax.axis_index("c")
    lanes = lax.iota(jnp.int32, 16)
    ones = jnp.ones((16,), jnp.int32)
    zeros = jnp.zeros((16,), jnp.int32)

    pltpu.sync_copy(ts_hbm, ts_buf)

    for sub in range(2):
        b = wid * 2 + sub
        prob_buf[_vec16(NVEC - 2)] = jnp.zeros((16,), jnp.float32)
        prob_buf[_vec16(NVEC - 1)] = jnp.zeros((16,), jnp.float32)
        pltpu.sync_copy(prob_hbm.at[b], prob_buf.at[pl.ds(0, N)])
        pltpu.sync_copy(boxes_hbm.at[b], boxf)

        def zh(i, _):
            hist[_vec16(i)] = zeros
            return 0

        lax.fori_loop(0, HBINS // 16, zh, 0, unroll=8)

        def hb1(i, m):
            v = prob_buf[_vec16(i)]
            bits = lax.bitcast_convert_type(v, jnp.int32)
            plsc.addupdate_scatter(hist, [lax.shift_right_logical(bits, 16)], ones)
            return jnp.maximum(m, v)

        maxv = lax.fori_loop(0, NVEC, hb1, jnp.zeros((16,), jnp.float32),
                             unroll=8)
        maxbin = jnp.max(lax.shift_right_logical(
            lax.bitcast_convert_type(maxv, jnp.int32), 16))

        bin1, above1, h1 = _scan_top(hist, lax.shift_right_logical(maxbin, 4),
                                     jnp.int32(NSEL))

        def sub_hist(pshift, pval, oshift):
            lax.fori_loop(0, 16, zh, 0)

            def hb2(i, _):
                v = prob_buf[_vec16(i)]
                bits = lax.bitcast_convert_type(v, jnp.int32)
                sel = lax.shift_right_logical(bits, pshift) == pval
                sbin = lax.shift_right_logical(bits, oshift) & 0xFF
                plsc.addupdate_scatter(hist, [sbin], ones, mask=sel)
                return 0

            lax.fori_loop(0, NVEC, hb2, 0, unroll=4)

        def case_a():
            return bin1 << 16, jnp.int32(BIG)

        def case_bc():
            sub_hist(16, bin1, 8)
            bin2, above2, h2 = _scan_top(hist, jnp.int32(15), NSEL - above1)

            def case_b():
                return (bin1 << 16) | (bin2 << 8), jnp.int32(BIG)

            def case_c():
                sub_hist(8, (bin1 << 8) | bin2, 0)
                bin3, above3, _ = _scan_top(hist, jnp.int32(15),
                                            NSEL - above1 - above2)
                cnt_gt = above1 + above2 + above3
                return (bin1 << 16) | (bin2 << 8) | bin3, NSEL - cnt_gt

            return lax.cond(above1 + above2 + h2 <= CAP, case_b, case_c)

        tlow_bits, need_eq = lax.cond(above1 + h1 <= CAP, case_a, case_bc)
        tlow_f = lax.bitcast_convert_type(jnp.broadcast_to(tlow_bits, (16,)),
                                          jnp.float32)

        for i in range(CBUF // 16):
            cidx[_vec16(i)] = jnp.full((16,), NPAD - 1, jnp.int32)

        def cb_fast(i, g):
            v = prob_buf[_vec16(i)]
            keep = v >= tlow_f
            plsc.store_compressed(cidx.at[pl.ds(g, 16)], i * 16 + lanes,
                                  mask=keep)
            return g + jnp.sum(keep.astype(jnp.int32))

        def cb_slow(i, carry):
            g, e = carry
            v = prob_buf[_vec16(i)]
            m_gt = v > tlow_f
            m_eq = v == tlow_f
            pfx = plsc.cumsum(m_eq.astype(jnp.int32))
            keep = jnp.logical_or(
                m_gt, jnp.logical_and(m_eq, (e + pfx) <= need_eq))
            plsc.store_compressed(cidx.at[pl.ds(g, 16)], i * 16 + lanes,
                                  mask=keep)
            return (g + jnp.sum(keep.astype(jnp.int32)),
                    e + jnp.sum(m_eq.astype(jnp.int32)))

        def compact_fast():
            return lax.fori_loop(0, NVEC, cb_fast, jnp.int32(0), unroll=4)

        def compact_slow():
            g, _ = lax.fori_loop(0, NVEC, cb_slow,
                                 (jnp.int32(0), jnp.int32(0)))
            return g

        g = lax.cond(need_eq >= BIG, compact_fast, compact_slow)
        gv = (g + 15) // 16

        def fill_keys(i, _):
            ckey[_vec16(i)] = plsc.load_gather(prob_buf, [cidx[_vec16(i)]])
            return 0

        lax.fori_loop(0, CBUF // 16, fill_keys, 0, unroll=4)

        def zq(r, _):
            qidx_st[_vec16(r)] = zeros
            return 0

        lax.fori_loop(0, NSEL_PAD // 16, zq, 0)

        def rank_outer(ev, _):
            key_e = ckey[_vec16(ev)]
            idx_e = cidx[_vec16(ev)]
            pos_e = ev * 16 + lanes

            def rchunk(jc, r):
                kj = ckey[_vec16(jc)]
                jbase = jc * 16
                for l in range(16):
                    kb = kj[jnp.full((16,), l, jnp.int32)]
                    gt = kb > key_e
                    eqb = jnp.logical_and(kb == key_e, jbase + l < pos_e)
                    r = r + jnp.logical_or(gt, eqb).astype(jnp.int32)
                return r

            rank = lax.fori_loop(0, gv, rchunk, zeros)
            msk = rank < NSEL
            plsc.store_scatter(scores_st, [rank], key_e, mask=msk)
            plsc.store_scatter(labels_st, [rank], idx_e % C, mask=msk)
            plsc.store_scatter(qidx_st, [rank], idx_e // C, mask=msk)
            return 0

        lax.fori_loop(0, gv, rank_outer, 0)

        img_h = plsc.load_gather(ts_buf, [jnp.broadcast_to(b, (16,)), zeros])
        img_w = plsc.load_gather(ts_buf, [jnp.broadcast_to(b, (16,)), ones])

        def bx(r, _):
            q = qidx_st[_vec16(r)]
            cx = plsc.load_gather(boxf, [q, zeros])
            cy = plsc.load_gather(boxf, [q, ones])
            w = plsc.load_gather(boxf, [q, ones + 1])
            h = plsc.load_gather(boxf, [q, ones + 2])
            rows = r * 16 + lanes
            plsc.store_scatter(boxes_st, [rows, zeros], (cx - 0.5 * w) * img_w)
            plsc.store_scatter(boxes_st, [rows, ones], (cy - 0.5 * h) * img_h)
            plsc.store_scatter(boxes_st, [rows, ones + 1], (cx + 0.5 * w) * img_w)
            plsc.store_scatter(boxes_st, [rows, ones + 2], (cy + 0.5 * h) * img_h)
            return 0

        lax.fori_loop(0, NSEL_PAD // 16, bx, 0)

        pltpu.sync_copy(boxes_st.at[pl.ds(0, NSEL)], out_boxes.at[b])
        pltpu.sync_copy(scores_st, out_scores.at[b])
        pltpu.sync_copy(labels_st, out_labels.at[b])


@jax.jit
def kernel(pred_logits, pred_boxes, target_sizes):
    prob = jax.nn.sigmoid(pred_logits).reshape(B, N)

    mesh = plsc.VectorSubcoreMesh(core_axis_name="c", subcore_axis_name="s",
                                  num_cores=2, num_subcores=16)
    run = pl.kernel(
        _body,
        out_type=[
            jax.ShapeDtypeStruct((B, NSEL, 4), jnp.float32),
            jax.ShapeDtypeStruct((B, NSEL_PAD), jnp.float32),
            jax.ShapeDtypeStruct((B, NSEL_PAD), jnp.int32),
        ],
        mesh=mesh,
        compiler_params=pltpu.CompilerParams(needs_layout_passes=False,
                                             use_tc_tiling_on_sc=False),
        scratch_types=[
            pltpu.VMEM((NPAD,), jnp.float32),
            pltpu.VMEM((HBINS,), jnp.int32),
            pltpu.VMEM((CBUF,), jnp.float32),
            pltpu.VMEM((CBUF,), jnp.int32),
            pltpu.VMEM((Q, 4), jnp.float32),
            pltpu.VMEM((B, 2), jnp.float32),
            pltpu.VMEM((NSEL_PAD, 4), jnp.float32),
            pltpu.VMEM((NSEL_PAD,), jnp.float32),
            pltpu.VMEM((NSEL_PAD,), jnp.int32),
            pltpu.VMEM((NSEL_PAD,), jnp.int32),
        ],
    )
    boxes, scores, labels = run(prob, pred_boxes, target_sizes)
    return boxes, scores[:, :NSEL], labels[:, :NSEL]


# speedup vs baseline: 1.1904x; 1.1901x over previous
import jax
import jax.numpy as jnp
from jax import lax
from jax.experimental import pallas as pl
from jax.experimental.pallas import tpu as pltpu
from jax.experimental.pallas import tpu_sc as plsc

B = 64
Q = 900
C = 91
N = Q * C
NPAD = 81920
NVEC = NPAD // 16
NSEL = 300
NSEL_PAD = 304
CAP = 512
CBUF = CAP + 16
HBINS = 16384
BIG = 1 << 20


def _vec16(i):
    return pl.ds(pl.multiple_of(i * 16, 16), 16)


def _scan_top(hist, start_chunk, need):
    def cond(c):
        cum, _ = c
        return cum < need

    def body(c):
        cum, j = c
        h = hist[_vec16(j)]
        return cum + jnp.sum(h), j - 1

    cum, j = lax.while_loop(cond, body, (jnp.int32(0), start_chunk))
    jc = j + 1
    h = hist[_vec16(jc)]
    base = cum - jnp.sum(h)
    rev = lax.rev(h, (0,))
    cs = plsc.cumsum(rev)
    lanes = lax.iota(jnp.int32, 16)
    f = jnp.max(plsc.all_reduce_ffs((base + cs) >= need) * jnp.ones((16,), jnp.int32))
    hb = jnp.sum(jnp.where(lanes == f, rev, 0))
    csf = jnp.sum(jnp.where(lanes == f, cs, 0))
    return jc * 16 + 15 - f, base + csf - hb, hb


def _body(prob_hbm, boxes_hbm, ts_hbm, out_boxes, out_scores, out_labels,
          prob_buf, hist, ckey, cidx, boxf, ts_buf,
          boxes_st, scores_st, labels_st, qidx_st):
    wid = lax.axis_index("s") * 2 + lax.axis_index("c")
    lanes = lax.iota(jnp.int32, 16)
    ones = jnp.ones((16,), jnp.int32)
    zeros = jnp.zeros((16,), jnp.int32)

    pltpu.sync_copy(ts_hbm, ts_buf)

    for sub in range(2):
        b = wid * 2 + sub
        pltpu.sync_copy(prob_hbm.at[b], prob_buf)
        pltpu.sync_copy(boxes_hbm.at[b], boxf)

        def zh(i, _):
            hist[_vec16(i)] = zeros
            return 0

        with jax.named_scope("ph_zero"):
            lax.fori_loop(0, HBINS // 16, zh, 0, unroll=8)

        def hb1(i, m):
            v = prob_buf[_vec16(i)]
            bits = lax.bitcast_convert_type(v, jnp.int32)
            plsc.addupdate_scatter(hist, [lax.shift_right_logical(bits, 16)], ones)
            return jnp.maximum(m, v)

        with jax.named_scope("ph_hist"):
            maxv = lax.fori_loop(0, NVEC, hb1, jnp.zeros((16,), jnp.float32),
                                 unroll=8)
        maxbin = jnp.max(lax.shift_right_logical(
            lax.bitcast_convert_type(maxv, jnp.int32), 16))

        bin1, above1, h1 = _scan_top(hist, lax.shift_right_logical(maxbin, 4),
                                     jnp.int32(NSEL))

        def sub_hist(pshift, pval, oshift):
            lax.fori_loop(0, 16, zh, 0)

            def hb2(i, _):
                v = prob_buf[_vec16(i)]
                bits = lax.bitcast_convert_type(v, jnp.int32)
                sel = lax.shift_right_logical(bits, pshift) == pval
                sbin = lax.shift_right_logical(bits, oshift) & 0xFF
                plsc.addupdate_scatter(hist, [sbin], ones, mask=sel)
                return 0

            lax.fori_loop(0, NVEC, hb2, 0, unroll=4)

        def case_a():
            return bin1 << 16, jnp.int32(BIG)

        def case_bc():
            sub_hist(16, bin1, 8)
            bin2, above2, h2 = _scan_top(hist, jnp.int32(15), NSEL - above1)

            def case_b():
                return (bin1 << 16) | (bin2 << 8), jnp.int32(BIG)

            def case_c():
                sub_hist(8, (bin1 << 8) | bin2, 0)
                bin3, above3, _ = _scan_top(hist, jnp.int32(15),
                                            NSEL - above1 - above2)
                cnt_gt = above1 + above2 + above3
                return (bin1 << 16) | (bin2 << 8) | bin3, NSEL - cnt_gt

            return lax.cond(above1 + above2 + h2 <= CAP, case_b, case_c)

        tlow_bits, need_eq = lax.cond(above1 + h1 <= CAP, case_a, case_bc)
        tlow_f = lax.bitcast_convert_type(jnp.broadcast_to(tlow_bits, (16,)),
                                          jnp.float32)

        for i in range(CBUF // 16):
            cidx[_vec16(i)] = jnp.full((16,), NPAD - 1, jnp.int32)

        def cb_fast(i, g):
            v = prob_buf[_vec16(i)]
            keep = v >= tlow_f
            plsc.store_compressed(cidx.at[pl.ds(g, 16)], i * 16 + lanes,
                                  mask=keep)
            return g + jnp.sum(keep.astype(jnp.int32))

        def cb_slow(i, carry):
            g, e = carry
            v = prob_buf[_vec16(i)]
            m_gt = v > tlow_f
            m_eq = v == tlow_f
            pfx = plsc.cumsum(m_eq.astype(jnp.int32))
            keep = jnp.logical_or(
                m_gt, jnp.logical_and(m_eq, (e + pfx) <= need_eq))
            plsc.store_compressed(cidx.at[pl.ds(g, 16)], i * 16 + lanes,
                                  mask=keep)
            return (g + jnp.sum(keep.astype(jnp.int32)),
                    e + jnp.sum(m_eq.astype(jnp.int32)))

        def compact_fast():
            return lax.fori_loop(0, NVEC, cb_fast, jnp.int32(0), unroll=4)

        def compact_slow():
            g, _ = lax.fori_loop(0, NVEC, cb_slow,
                                 (jnp.int32(0), jnp.int32(0)))
            return g

        with jax.named_scope("ph_compact"):
            g = lax.cond(need_eq >= BIG, compact_fast, compact_slow)
        gv = (g + 15) // 16

        def fill_keys(i, _):
            ckey[_vec16(i)] = plsc.load_gather(prob_buf, [cidx[_vec16(i)]])
            return 0

        lax.fori_loop(0, CBUF // 16, fill_keys, 0, unroll=4)

        def zq(r, _):
            qidx_st[_vec16(r)] = zeros
            return 0

        lax.fori_loop(0, NSEL_PAD // 16, zq, 0)

        def rank_outer(ev, _):
            key_e = ckey[_vec16(ev)]
            idx_e = cidx[_vec16(ev)]
            pos_e = ev * 16 + lanes

            def rchunk(jc, r):
                kj = ckey[_vec16(jc)]
                jbase = jc * 16
                for l in range(16):
                    kb = kj[jnp.full((16,), l, jnp.int32)]
                    gt = kb > key_e
                    eqb = jnp.logical_and(kb == key_e, jbase + l < pos_e)
                    r = r + jnp.logical_or(gt, eqb).astype(jnp.int32)
                return r

            rank = lax.fori_loop(0, gv, rchunk, zeros)
            msk = rank < NSEL
            plsc.store_scatter(scores_st, [rank], key_e, mask=msk)
            plsc.store_scatter(labels_st, [rank], idx_e % C, mask=msk)
            plsc.store_scatter(qidx_st, [rank], idx_e // C, mask=msk)
            return 0

        with jax.named_scope("ph_rank"):
            lax.fori_loop(0, gv, rank_outer, 0)

        img_h = plsc.load_gather(ts_buf, [jnp.broadcast_to(b, (16,)), zeros])
        img_w = plsc.load_gather(ts_buf, [jnp.broadcast_to(b, (16,)), ones])

        def bx(r, _):
            q = qidx_st[_vec16(r)]
            cx = plsc.load_gather(boxf, [q, zeros])
            cy = plsc.load_gather(boxf, [q, ones])
            w = plsc.load_gather(boxf, [q, ones + 1])
            h = plsc.load_gather(boxf, [q, ones + 2])
            rows = r * 16 + lanes
            plsc.store_scatter(boxes_st, [rows, zeros], (cx - 0.5 * w) * img_w)
            plsc.store_scatter(boxes_st, [rows, ones], (cy - 0.5 * h) * img_h)
            plsc.store_scatter(boxes_st, [rows, ones + 1], (cx + 0.5 * w) * img_w)
            plsc.store_scatter(boxes_st, [rows, ones + 2], (cy + 0.5 * h) * img_h)
            return 0

        with jax.named_scope("ph_boxes"):
            lax.fori_loop(0, NSEL_PAD // 16, bx, 0)

        pltpu.sync_copy(boxes_st.at[pl.ds(0, NSEL)], out_boxes.at[b])
        pltpu.sync_copy(scores_st, out_scores.at[b])
        pltpu.sync_copy(labels_st, out_labels.at[b])


@jax.jit
def kernel(pred_logits, pred_boxes, target_sizes):
    prob = jax.nn.sigmoid(pred_logits).reshape(B, N)
    prob = jnp.pad(prob, ((0, 0), (0, NPAD - N)))

    mesh = plsc.VectorSubcoreMesh(core_axis_name="c", subcore_axis_name="s",
                                  num_cores=2, num_subcores=16)
    run = pl.kernel(
        _body,
        out_type=[
            jax.ShapeDtypeStruct((B, NSEL, 4), jnp.float32),
            jax.ShapeDtypeStruct((B, NSEL_PAD), jnp.float32),
            jax.ShapeDtypeStruct((B, NSEL_PAD), jnp.int32),
        ],
        mesh=mesh,
        compiler_params=pltpu.CompilerParams(needs_layout_passes=False,
                                             use_tc_tiling_on_sc=False),
        scratch_types=[
            pltpu.VMEM((NPAD,), jnp.float32),
            pltpu.VMEM((HBINS,), jnp.int32),
            pltpu.VMEM((CBUF,), jnp.float32),
            pltpu.VMEM((CBUF,), jnp.int32),
            pltpu.VMEM((Q, 4), jnp.float32),
            pltpu.VMEM((B, 2), jnp.float32),
            pltpu.VMEM((NSEL_PAD, 4), jnp.float32),
            pltpu.VMEM((NSEL_PAD,), jnp.float32),
            pltpu.VMEM((NSEL_PAD,), jnp.int32),
            pltpu.VMEM((NSEL_PAD,), jnp.int32),
        ],
    )
    boxes, scores, labels = run(prob, pred_boxes, target_sizes)
    return boxes, scores[:, :NSEL], labels[:, :NSEL]


# speedup vs baseline: 1.5079x; 1.2668x over previous
import jax
import jax.numpy as jnp
from jax import lax
from jax.experimental import pallas as pl
from jax.experimental.pallas import tpu as pltpu
from jax.experimental.pallas import tpu_sc as plsc

B = 64
Q = 900
C = 91
N = Q * C
NPAD = 81920
NVEC = NPAD // 16
NSEL = 300
NSEL_PAD = 304
CAP = 512
CBUF = CAP + 16
HBINS = 16384
BIG = 1 << 20


def _vec16(i):
    return pl.ds(pl.multiple_of(i * 16, 16), 16)


def _scan_top(hist, start_chunk, need):
    def cond(c):
        cum, _ = c
        return cum < need

    def body(c):
        cum, j = c
        h = hist[_vec16(j)]
        return cum + jnp.sum(h), j - 1

    cum, j = lax.while_loop(cond, body, (jnp.int32(0), start_chunk))
    jc = j + 1
    h = hist[_vec16(jc)]
    base = cum - jnp.sum(h)
    rev = lax.rev(h, (0,))
    cs = plsc.cumsum(rev)
    lanes = lax.iota(jnp.int32, 16)
    f = jnp.max(plsc.all_reduce_ffs((base + cs) >= need) * jnp.ones((16,), jnp.int32))
    hb = jnp.sum(jnp.where(lanes == f, rev, 0))
    csf = jnp.sum(jnp.where(lanes == f, cs, 0))
    return jc * 16 + 15 - f, base + csf - hb, hb


def _body(prob_hbm, boxes_hbm, ts_hbm, out_boxes, out_scores, out_labels,
          prob_buf, hist, ckey, cidx, boxf, ts_buf,
          boxes_st, scores_st, labels_st, qidx_st):
    wid = lax.axis_index("s") * 2 + lax.axis_index("c")
    lanes = lax.iota(jnp.int32, 16)
    ones = jnp.ones((16,), jnp.int32)
    zeros = jnp.zeros((16,), jnp.int32)

    pltpu.sync_copy(ts_hbm, ts_buf)

    for sub in range(2):
        b = wid * 2 + sub
        pltpu.sync_copy(prob_hbm.at[b], prob_buf)
        pltpu.sync_copy(boxes_hbm.at[b], boxf)

        def zh(i, _):
            hist[_vec16(i)] = zeros
            return 0

        lax.fori_loop(0, HBINS // 16, zh, 0, unroll=8)

        def hb1(i, m):
            v = prob_buf[_vec16(i)]
            bits = lax.bitcast_convert_type(v, jnp.int32)
            plsc.addupdate_scatter(hist, [lax.shift_right_logical(bits, 16)], ones)
            return jnp.maximum(m, v)

        maxv = lax.fori_loop(0, NVEC, hb1, jnp.zeros((16,), jnp.float32),
                             unroll=8)
        maxbin = jnp.max(lax.shift_right_logical(
            lax.bitcast_convert_type(maxv, jnp.int32), 16))

        bin1, above1, h1 = _scan_top(hist, lax.shift_right_logical(maxbin, 4),
                                     jnp.int32(NSEL))

        def sub_hist(pshift, pval, oshift):
            lax.fori_loop(0, 16, zh, 0)

            def hb2(i, _):
                v = prob_buf[_vec16(i)]
                bits = lax.bitcast_convert_type(v, jnp.int32)
                sel = lax.shift_right_logical(bits, pshift) == pval
                sbin = lax.shift_right_logical(bits, oshift) & 0xFF
                plsc.addupdate_scatter(hist, [sbin], ones, mask=sel)
                return 0

            lax.fori_loop(0, NVEC, hb2, 0, unroll=4)

        def case_a():
            return bin1 << 16, jnp.int32(BIG)

        def case_bc():
            sub_hist(16, bin1, 8)
            bin2, above2, h2 = _scan_top(hist, jnp.int32(15), NSEL - above1)

            def case_b():
                return (bin1 << 16) | (bin2 << 8), jnp.int32(BIG)

            def case_c():
                sub_hist(8, (bin1 << 8) | bin2, 0)
                bin3, above3, _ = _scan_top(hist, jnp.int32(15),
                                            NSEL - above1 - above2)
                cnt_gt = above1 + above2 + above3
                return (bin1 << 16) | (bin2 << 8) | bin3, NSEL - cnt_gt

            return lax.cond(above1 + above2 + h2 <= CAP, case_b, case_c)

        tlow_bits, need_eq = lax.cond(above1 + h1 <= CAP, case_a, case_bc)
        tlow_f = lax.bitcast_convert_type(jnp.broadcast_to(tlow_bits, (16,)),
                                          jnp.float32)

        for i in range(CBUF // 16):
            cidx[_vec16(i)] = jnp.full((16,), NPAD - 1, jnp.int32)

        def cb_fast(i, g):
            v = prob_buf[_vec16(i)]
            keep = v >= tlow_f
            plsc.store_compressed(cidx.at[pl.ds(g, 16)], i * 16 + lanes,
                                  mask=keep)
            return g + jnp.sum(keep.astype(jnp.int32))

        def cb_slow(i, carry):
            g, e = carry
            v = prob_buf[_vec16(i)]
            m_gt = v > tlow_f
            m_eq = v == tlow_f
            pfx = plsc.cumsum(m_eq.astype(jnp.int32))
            keep = jnp.logical_or(
                m_gt, jnp.logical_and(m_eq, (e + pfx) <= need_eq))
            plsc.store_compressed(cidx.at[pl.ds(g, 16)], i * 16 + lanes,
                                  mask=keep)
            return (g + jnp.sum(keep.astype(jnp.int32)),
                    e + jnp.sum(m_eq.astype(jnp.int32)))

        def compact_fast():
            return lax.fori_loop(0, NVEC, cb_fast, jnp.int32(0), unroll=4)

        def compact_slow():
            g, _ = lax.fori_loop(0, NVEC, cb_slow,
                                 (jnp.int32(0), jnp.int32(0)))
            return g

        g = lax.cond(need_eq >= BIG, compact_fast, compact_slow)
        gv = (g + 15) // 16

        def fill_keys(i, _):
            ckey[_vec16(i)] = plsc.load_gather(prob_buf, [cidx[_vec16(i)]])
            return 0

        lax.fori_loop(0, CBUF // 16, fill_keys, 0, unroll=4)

        def zq(r, _):
            qidx_st[_vec16(r)] = zeros
            return 0

        lax.fori_loop(0, NSEL_PAD // 16, zq, 0)

        def rank_outer(ev, _):
            key_e = ckey[_vec16(ev)]
            idx_e = cidx[_vec16(ev)]

            def pair_sum(kj, cmp):
                ra = zeros
                rb = zeros
                for l in range(16):
                    kb = kj[jnp.full((16,), l, jnp.int32)]
                    t = cmp(kb, l).astype(jnp.int32)
                    if l % 2 == 0:
                        ra = ra + t
                    else:
                        rb = rb + t
                return ra + rb

            def chunk_ge(jc, r):
                return r + pair_sum(ckey[_vec16(jc)],
                                    lambda kb, l: kb >= key_e)

            def chunk_gt(jc, r):
                return r + pair_sum(ckey[_vec16(jc)],
                                    lambda kb, l: kb > key_e)

            r1 = lax.fori_loop(0, ev, chunk_ge, zeros)
            r2 = lax.fori_loop(ev + 1, gv, chunk_gt, r1)
            rank = r2 + pair_sum(key_e, lambda kb, l: jnp.logical_or(
                kb > key_e, jnp.logical_and(kb == key_e, l < lanes)))
            msk = rank < NSEL
            plsc.store_scatter(scores_st, [rank], key_e, mask=msk)
            plsc.store_scatter(labels_st, [rank], idx_e % C, mask=msk)
            plsc.store_scatter(qidx_st, [rank], idx_e // C, mask=msk)
            return 0

        lax.fori_loop(0, gv, rank_outer, 0)

        img_h = plsc.load_gather(ts_buf, [jnp.broadcast_to(b, (16,)), zeros])
        img_w = plsc.load_gather(ts_buf, [jnp.broadcast_to(b, (16,)), ones])

        def bx(r, _):
            q = qidx_st[_vec16(r)]
            cx = plsc.load_gather(boxf, [q, zeros])
            cy = plsc.load_gather(boxf, [q, ones])
            w = plsc.load_gather(boxf, [q, ones + 1])
            h = plsc.load_gather(boxf, [q, ones + 2])
            rows = r * 16 + lanes
            plsc.store_scatter(boxes_st, [rows, zeros], (cx - 0.5 * w) * img_w)
            plsc.store_scatter(boxes_st, [rows, ones], (cy - 0.5 * h) * img_h)
            plsc.store_scatter(boxes_st, [rows, ones + 1], (cx + 0.5 * w) * img_w)
            plsc.store_scatter(boxes_st, [rows, ones + 2], (cy + 0.5 * h) * img_h)
            return 0

        lax.fori_loop(0, NSEL_PAD // 16, bx, 0)

        pltpu.sync_copy(boxes_st.at[pl.ds(0, NSEL)], out_boxes.at[b])
        pltpu.sync_copy(scores_st, out_scores.at[b])
        pltpu.sync_copy(labels_st, out_labels.at[b])


@jax.jit
def kernel(pred_logits, pred_boxes, target_sizes):
    prob = jax.nn.sigmoid(pred_logits).reshape(B, N)
    prob = jnp.pad(prob, ((0, 0), (0, NPAD - N)))

    mesh = plsc.VectorSubcoreMesh(core_axis_name="c", subcore_axis_name="s",
                                  num_cores=2, num_subcores=16)
    run = pl.kernel(
        _body,
        out_type=[
            jax.ShapeDtypeStruct((B, NSEL, 4), jnp.float32),
            jax.ShapeDtypeStruct((B, NSEL_PAD), jnp.float32),
            jax.ShapeDtypeStruct((B, NSEL_PAD), jnp.int32),
        ],
        mesh=mesh,
        compiler_params=pltpu.CompilerParams(needs_layout_passes=False,
                                             use_tc_tiling_on_sc=False),
        scratch_types=[
            pltpu.VMEM((NPAD,), jnp.float32),
            pltpu.VMEM((HBINS,), jnp.int32),
            pltpu.VMEM((CBUF,), jnp.float32),
            pltpu.VMEM((CBUF,), jnp.int32),
            pltpu.VMEM((Q, 4), jnp.float32),
            pltpu.VMEM((B, 2), jnp.float32),
            pltpu.VMEM((NSEL_PAD, 4), jnp.float32),
            pltpu.VMEM((NSEL_PAD,), jnp.float32),
            pltpu.VMEM((NSEL_PAD,), jnp.int32),
            pltpu.VMEM((NSEL_PAD,), jnp.int32),
        ],
    )
    boxes, scores, labels = run(prob, pred_boxes, target_sizes)
    return boxes, scores[:, :NSEL], labels[:, :NSEL]
